# Initial kernel scaffold; baseline (speedup 1.0000x reference)
import jax
import jax.numpy as jnp
from jax.experimental import pallas as pl


def kernel(x, gemm_edges, W, b):
    raise NotImplementedError("write your pallas kernel here")


# speedup vs baseline: 4.5363x; 4.5363x over previous
import functools

import jax
import jax.numpy as jnp
from jax import lax
from jax.experimental import pallas as pl
from jax.experimental.pallas import tpu as pltpu
from jax.experimental.pallas import tpu_sc as plsc

E = 160000
C = 128
NC, NS = 2, 16
NW = NC * NS
CH = 128
NCHUNKS = E // CH
CHUNKS_PER_W = -(-NCHUNKS // NW)
TE = 640


def _sc_gather(xt, idx):
    mesh = plsc.VectorSubcoreMesh(
        core_axis_name="c", subcore_axis_name="s",
        num_cores=NC, num_subcores=NS)

    @functools.partial(
        pl.kernel,
        out_type=jax.ShapeDtypeStruct((4, E, C), jnp.float32),
        mesh=mesh,
        scratch_types=[
            pltpu.VMEM((CH,), jnp.int32),
            pltpu.VMEM((CH,), jnp.int32),
            pltpu.VMEM((CH,), jnp.int32),
            pltpu.VMEM((CH,), jnp.int32),
            pltpu.VMEM((CH, C), jnp.float32),
            pltpu.VMEM((CH, C), jnp.float32),
            pltpu.VMEM((CH, C), jnp.float32),
            pltpu.VMEM((CH, C), jnp.float32),
            pltpu.SemaphoreType.DMA,
        ],
    )
    def gather_kernel(xt_hbm, idx_hbm, g_hbm, i0, i1, i2, i3,
                      b0, b1, b2, b3, sem):
        idxb = (i0, i1, i2, i3)
        bufs = (b0, b1, b2, b3)
        wid = lax.axis_index("s") * NC + lax.axis_index("c")

        def body(i, carry):
            chunk = wid + i * NW

            @pl.when(chunk < NCHUNKS)
            def _():
                base = chunk * CH
                for k in range(4):
                    pltpu.sync_copy(idx_hbm.at[k, pl.ds(base, CH)], idxb[k])
                descs = [pltpu.async_copy(xt_hbm.at[idxb[k]], bufs[k], sem)
                         for k in range(4)]
                for d in descs:
                    d.wait()
                for k in range(4):
                    pltpu.sync_copy(bufs[k], g_hbm.at[k, pl.ds(base, CH)])

            return carry

        lax.fori_loop(0, CHUNKS_PER_W, body, 0)

    return gather_kernel(xt, idx)


def _tc_body(xt_ref, g_ref, wt_ref, b_ref, out_ref):
    y1 = g_ref[0]
    y2 = g_ref[1]
    y3 = g_ref[2]
    y4 = g_ref[3]
    s1 = y1 + y3
    s2 = y2 + y4
    d1 = jnp.abs(y1 - y3)
    d2 = jnp.abs(y2 - y4)
    acc = jnp.dot(xt_ref[...], wt_ref[0], preferred_element_type=jnp.float32)
    acc = acc + jnp.dot(s1, wt_ref[1], preferred_element_type=jnp.float32)
    acc = acc + jnp.dot(s2, wt_ref[2], preferred_element_type=jnp.float32)
    acc = acc + jnp.dot(d1, wt_ref[3], preferred_element_type=jnp.float32)
    acc = acc + jnp.dot(d2, wt_ref[4], preferred_element_type=jnp.float32)
    out_ref[...] = acc + b_ref[...]


def _tc_conv(xt, g, wt, b_row):
    return pl.pallas_call(
        _tc_body,
        grid=(E // TE,),
        in_specs=[
            pl.BlockSpec((TE, C), lambda i: (i, 0)),
            pl.BlockSpec((4, TE, C), lambda i: (0, i, 0)),
            pl.BlockSpec((5, C, C), lambda i: (0, 0, 0)),
            pl.BlockSpec((1, C), lambda i: (0, 0)),
        ],
        out_specs=pl.BlockSpec((TE, C), lambda i: (i, 0)),
        out_shape=jax.ShapeDtypeStruct((E, C), jnp.float32),
    )(xt, g, wt, b_row)


def kernel(x, gemm_edges, W, b):
    xt = x[0, :, :, 0].T
    idx = gemm_edges[0].astype(jnp.int32).T
    wt = jnp.transpose(W[:, :, 0, :], (2, 1, 0))
    g = _sc_gather(xt, idx)
    out_t = _tc_conv(xt, g, wt, b[None, :])
    return out_t.T[None, :, :, None]
